# baseline (speedup 1.0000x reference)
import functools

import jax
import jax.numpy as jnp
from jax.experimental import pallas as pl
from jax.experimental.pallas import tpu as pltpu

NUM_EXPERTS = 64
TOP_K = 8
TB = 512
HB = 256


def _topk8(vals, iota):
    scores = []
    idxs = []
    for _ in range(TOP_K):
        m = jnp.max(vals, axis=-1, keepdims=True)
        i = jnp.argmax(vals, axis=-1, keepdims=True).astype(jnp.int32)
        scores.append(m)
        idxs.append(i)
        vals = jnp.where(iota == i, -jnp.inf, vals)
    return jnp.concatenate(scores, axis=-1), jnp.concatenate(idxs, axis=-1)


def _router_block(nblk, x_ref, wt_ref, scores_ref, idx_ref, scr_ref):
    i = pl.program_id(0)
    slot = jax.lax.rem(i, 2)

    @pl.when(i < nblk)
    def _matmul():
        scr_ref[slot] = jnp.dot(
            x_ref[...], wt_ref[...], preferred_element_type=jnp.float32)

    @pl.when(i > 0)
    def _select():
        iota = jax.lax.broadcasted_iota(jnp.int32, (TB, NUM_EXPERTS), 1)
        s, ix = _topk8(scr_ref[1 - slot], iota)
        scores_ref[...] = s
        idx_ref[...] = ix


@functools.partial(jax.jit, static_argnames=())
def kernel(hidden_states, W):
    tokens, hidden = hidden_states.shape
    wt = W.T
    nblk = tokens // TB
    scores, idx = pl.pallas_call(
        functools.partial(_router_block, nblk),
        grid=(nblk + 1,),
        in_specs=[
            pl.BlockSpec((TB, hidden), lambda i: (jnp.minimum(i, nblk - 1), 0)),
            pl.BlockSpec((hidden, NUM_EXPERTS), lambda i: (0, 0)),
        ],
        out_specs=[
            pl.BlockSpec((TB, TOP_K), lambda i: (jnp.maximum(i - 1, 0), 0)),
            pl.BlockSpec((TB, TOP_K), lambda i: (jnp.maximum(i - 1, 0), 0)),
        ],
        out_shape=[
            jax.ShapeDtypeStruct((tokens, TOP_K), jnp.float32),
            jax.ShapeDtypeStruct((tokens, TOP_K), jnp.int32),
        ],
        scratch_shapes=[pltpu.VMEM((2, TB, NUM_EXPERTS), jnp.float32)],
    )(hidden_states, wt)
    return scores, idx


# speedup vs baseline: 1.1274x; 1.1274x over previous
import functools

import jax
import jax.numpy as jnp
from jax.experimental import pallas as pl
from jax.experimental.pallas import tpu as pltpu

NUM_EXPERTS = 64
TOP_K = 8
TB = 512
HB = 256


def _topk8(vals, iota):
    scores = []
    idxs = []
    for _ in range(TOP_K):
        m = jnp.max(vals, axis=-1, keepdims=True)
        i = jnp.argmax(vals, axis=-1, keepdims=True).astype(jnp.int32)
        scores.append(m)
        idxs.append(i)
        vals = jnp.where(iota == i, -jnp.inf, vals)
    return jnp.concatenate(scores, axis=-1), jnp.concatenate(idxs, axis=-1)


def _router_block(nblk, x_ref, wt_ref, scores_ref, idx_ref, scr_ref):
    i = pl.program_id(0)
    slot = jax.lax.rem(i, 2)
    iota = jax.lax.broadcasted_iota(jnp.int32, (TB, NUM_EXPERTS), 1)
    s, ix = _topk8(scr_ref[1 - slot], iota)
    scores_ref[...] = s
    idx_ref[...] = ix
    scr_ref[slot] = jnp.dot(
        x_ref[...], wt_ref[...], preferred_element_type=jnp.float32)


@functools.partial(jax.jit, static_argnames=())
def kernel(hidden_states, W):
    tokens, hidden = hidden_states.shape
    wt = W.T
    nblk = tokens // TB
    scores, idx = pl.pallas_call(
        functools.partial(_router_block, nblk),
        grid=(nblk + 1,),
        in_specs=[
            pl.BlockSpec((TB, hidden), lambda i: (jnp.minimum(i, nblk - 1), 0)),
            pl.BlockSpec((hidden, NUM_EXPERTS), lambda i: (0, 0)),
        ],
        out_specs=[
            pl.BlockSpec((TB, TOP_K), lambda i: (jnp.maximum(i - 1, 0), 0)),
            pl.BlockSpec((TB, TOP_K), lambda i: (jnp.maximum(i - 1, 0), 0)),
        ],
        out_shape=[
            jax.ShapeDtypeStruct((tokens, TOP_K), jnp.float32),
            jax.ShapeDtypeStruct((tokens, TOP_K), jnp.int32),
        ],
        scratch_shapes=[pltpu.VMEM((2, TB, NUM_EXPERTS), jnp.float32)],
    )(hidden_states, wt)
    return scores, idx
